# Initial kernel scaffold; baseline (speedup 1.0000x reference)
import jax
import jax.numpy as jnp
from jax.experimental import pallas as pl


def kernel(input, embedding):
    raise NotImplementedError("write your pallas kernel here")


# speedup vs baseline: 3.3889x; 3.3889x over previous
import jax
import jax.numpy as jnp
from jax.experimental import pallas as pl

_EMB = 32
_NUM = 1024
_B = 8
_S = 8 * 32 * 32
_SB = 1024
_NBLK = _S // _SB


def _vq_kernel(x_ref, e_ref, q_ref, ind_ref, acc_ref):
    x = x_ref[0]
    e = e_ref[...]

    e2 = jnp.sum(e * e, axis=0)[:, None]
    prod = jax.lax.dot_general(e, x, (((0,), (0,)), ((), ())),
                               preferred_element_type=jnp.float32)
    scores = e2 - 2.0 * prod

    idx = jnp.argmin(scores, axis=0).astype(jnp.int32)
    ind_ref[0, 0, 0] = idx

    onehot = (jax.lax.broadcasted_iota(jnp.int32, (_NUM, _SB), 0)
              == idx[None, :]).astype(jnp.float32)
    q = jax.lax.dot_general(e, onehot, (((1,), (0,)), ((), ())),
                            preferred_element_type=jnp.float32)
    qz = x + (q - x)
    q_ref[0] = qz

    @pl.when((pl.program_id(0) == 0) & (pl.program_id(1) == 0))
    def _():
        acc_ref[...] = jnp.zeros_like(acc_ref)
    acc_ref[...] += jnp.sum((q - x) ** 2).reshape(1, 1)


def kernel(input, embedding):
    x = input.reshape(_B, _EMB, _S)

    quant, ind, acc = pl.pallas_call(
        _vq_kernel,
        grid=(_B, _NBLK),
        in_specs=[
            pl.BlockSpec((1, _EMB, _SB), lambda b, s: (b, 0, s)),
            pl.BlockSpec((_EMB, _NUM), lambda b, s: (0, 0)),
        ],
        out_specs=[
            pl.BlockSpec((1, _EMB, _SB), lambda b, s: (b, 0, s)),
            pl.BlockSpec((1, 1, 1, _SB), lambda b, s: (b, s, 0, 0)),
            pl.BlockSpec((1, 1), lambda b, s: (0, 0)),
        ],
        out_shape=[
            jax.ShapeDtypeStruct((_B, _EMB, _S), jnp.float32),
            jax.ShapeDtypeStruct((_B, _NBLK, 1, _SB), jnp.int32),
            jax.ShapeDtypeStruct((1, 1), jnp.float32),
        ],
    )(x, embedding)

    quantize = quant.reshape(input.shape)
    diff = (acc[0, 0] / (_B * _S * _EMB)).astype(jnp.float32)
    embedding_ind = ind.reshape(_B, 8, 32, 32)
    return quantize, diff, embedding_ind


# speedup vs baseline: 3.4731x; 1.0249x over previous
import jax
import jax.numpy as jnp
from jax.experimental import pallas as pl

_EMB = 32
_NUM = 1024
_B = 8
_S = 8 * 32 * 32
_SB = 1024
_NBLK = _S // _SB


def _vq_kernel(x_ref, e_ref, q_ref, ind_ref, acc_ref):
    x = x_ref[0]
    e = e_ref[...]

    e2 = jnp.sum(e * e, axis=0)[None, :]
    e_aug = jnp.concatenate([-2.0 * e, e2], axis=0)
    ones = jnp.ones((1, _SB), jnp.float32)
    x_aug = jnp.concatenate([x, ones], axis=0)
    scores = jax.lax.dot_general(e_aug, x_aug, (((0,), (0,)), ((), ())),
                                 preferred_element_type=jnp.float32)

    m = jnp.min(scores, axis=0)[None, :]
    mask = (scores <= m).astype(jnp.float32)

    iota = jax.lax.broadcasted_iota(jnp.int32, (1, _NUM), 1).astype(jnp.float32)
    g_aug = jnp.concatenate([e, iota], axis=0)
    out = jax.lax.dot_general(g_aug, mask, (((1,), (0,)), ((), ())),
                              preferred_element_type=jnp.float32)
    q = out[:_EMB]
    ind_ref[0, 0, 0] = out[_EMB].astype(jnp.int32)

    q_ref[0] = x + (q - x)

    @pl.when((pl.program_id(0) == 0) & (pl.program_id(1) == 0))
    def _():
        acc_ref[...] = jnp.zeros_like(acc_ref)
    acc_ref[...] += jnp.sum((q - x) ** 2).reshape(1, 1)


def kernel(input, embedding):
    x = input.reshape(_B, _EMB, _S)

    quant, ind, acc = pl.pallas_call(
        _vq_kernel,
        grid=(_B, _NBLK),
        in_specs=[
            pl.BlockSpec((1, _EMB, _SB), lambda b, s: (b, 0, s)),
            pl.BlockSpec((_EMB, _NUM), lambda b, s: (0, 0)),
        ],
        out_specs=[
            pl.BlockSpec((1, _EMB, _SB), lambda b, s: (b, 0, s)),
            pl.BlockSpec((1, 1, 1, _SB), lambda b, s: (b, s, 0, 0)),
            pl.BlockSpec((1, 1), lambda b, s: (0, 0)),
        ],
        out_shape=[
            jax.ShapeDtypeStruct((_B, _EMB, _S), jnp.float32),
            jax.ShapeDtypeStruct((_B, _NBLK, 1, _SB), jnp.int32),
            jax.ShapeDtypeStruct((1, 1), jnp.float32),
        ],
    )(x, embedding)

    quantize = quant.reshape(input.shape)
    diff = (acc[0, 0] / (_B * _S * _EMB)).astype(jnp.float32)
    embedding_ind = ind.reshape(_B, 8, 32, 32)
    return quantize, diff, embedding_ind


# speedup vs baseline: 3.6106x; 1.0396x over previous
import jax
import jax.numpy as jnp
from jax.experimental import pallas as pl

_EMB = 32
_NUM = 1024
_B = 8
_S = 8 * 32 * 32
_SB = 1024
_NBLK = _S // _SB


def _vq_kernel(x_ref, e_ref, q_ref, ind_ref, acc_ref):
    x = x_ref[0]
    e = e_ref[...]

    e2 = jnp.sum(e * e, axis=0)[:, None]
    prod = jax.lax.dot_general(e, x, (((0,), (0,)), ((), ())),
                               preferred_element_type=jnp.float32)
    scores = e2 - 2.0 * prod

    m = jnp.min(scores, axis=0)[None, :]
    mask = (scores <= m).astype(jnp.float32)

    iota = jax.lax.broadcasted_iota(jnp.int32, (1, _NUM), 1).astype(jnp.float32)
    g_aug = jnp.concatenate([e, iota], axis=0)
    out = jax.lax.dot_general(g_aug, mask, (((1,), (0,)), ((), ())),
                              preferred_element_type=jnp.float32)
    q = out[:_EMB]
    ind_ref[0, 0, 0] = out[_EMB].astype(jnp.int32)

    q_ref[0] = x + (q - x)

    @pl.when((pl.program_id(0) == 0) & (pl.program_id(1) == 0))
    def _():
        acc_ref[...] = jnp.zeros_like(acc_ref)
    acc_ref[...] += jnp.sum((q - x) ** 2).reshape(1, 1)


def kernel(input, embedding):
    x = input.reshape(_B, _EMB, _S)

    quant, ind, acc = pl.pallas_call(
        _vq_kernel,
        grid=(_B, _NBLK),
        in_specs=[
            pl.BlockSpec((1, _EMB, _SB), lambda b, s: (b, 0, s)),
            pl.BlockSpec((_EMB, _NUM), lambda b, s: (0, 0)),
        ],
        out_specs=[
            pl.BlockSpec((1, _EMB, _SB), lambda b, s: (b, 0, s)),
            pl.BlockSpec((1, 1, 1, _SB), lambda b, s: (b, s, 0, 0)),
            pl.BlockSpec((1, 1), lambda b, s: (0, 0)),
        ],
        out_shape=[
            jax.ShapeDtypeStruct((_B, _EMB, _S), jnp.float32),
            jax.ShapeDtypeStruct((_B, _NBLK, 1, _SB), jnp.int32),
            jax.ShapeDtypeStruct((1, 1), jnp.float32),
        ],
    )(x, embedding)

    quantize = quant.reshape(input.shape)
    diff = (acc[0, 0] / (_B * _S * _EMB)).astype(jnp.float32)
    embedding_ind = ind.reshape(_B, 8, 32, 32)
    return quantize, diff, embedding_ind
